# baseline (device time: 382509 ns/iter reference)
import jax
import jax.numpy as jnp
from jax import lax
from jax.experimental import pallas as pl
from jax.experimental.pallas import tpu as pltpu

NC = 16


def kernel(x):
    m, n = x.shape
    qsize = m // 4
    ch = qsize // NC
    nc2 = NC // 2

    def body(x_ref, out_ref, recv_ref, xq_ref,
             p1s, p1r, dys, dyr, dzs, dzr, fys, fyr, fzs, fzr, sts, xls):
        my_x = lax.axis_index("x")
        my_y = lax.axis_index("y")
        my_z = lax.axis_index("z")
        r = 2 * my_y + my_z
        ry = r ^ 2
        rz = r ^ 1
        diag = r ^ 3
        base = r * qsize
        x_nbr = (1 - my_x, my_y, my_z)
        y_nbr = (my_x, 1 - my_y, my_z)
        z_nbr = (my_x, my_y, 1 - my_z)

        barrier_sem = pltpu.get_barrier_semaphore()
        for nbr in (x_nbr, y_nbr, z_nbr):
            pl.semaphore_signal(
                barrier_sem, inc=1, device_id=nbr,
                device_id_type=pl.DeviceIdType.MESH,
            )
        pl.semaphore_wait(barrier_sem, 3)

        xq_load = pltpu.make_async_copy(
            x_ref.at[pl.ds(base, qsize)], xq_ref, xls)
        xq_load.start()

        p1 = []
        for c in range(NC):
            d = pltpu.make_async_remote_copy(
                src_ref=x_ref.at[pl.ds(base + c * ch, ch)],
                dst_ref=recv_ref.at[pl.ds(c * ch, ch)],
                send_sem=p1s.at[c],
                recv_sem=p1r.at[c],
                device_id=x_nbr,
                device_id_type=pl.DeviceIdType.MESH,
            )
            d.start()
            p1.append(d)

        def recv_desc(qidx, c, sems_r):
            sl = pl.ds(qidx * qsize + c * ch, ch)
            return pltpu.make_async_remote_copy(
                src_ref=out_ref.at[sl], dst_ref=out_ref.at[sl],
                send_sem=p1s.at[0],
                recv_sem=sems_r,
                device_id=x_nbr,
                device_id_type=pl.DeviceIdType.MESH,
            )

        fwd = []

        def process_arrival(a):
            recv_desc(rz, a, dzr.at[a]).wait_recv()
            if a < nc2:
                sl = pl.ds(rz * qsize + a * ch, ch)
                d = pltpu.make_async_remote_copy(
                    src_ref=out_ref.at[sl], dst_ref=out_ref.at[sl],
                    send_sem=fys.at[a], recv_sem=fyr.at[a],
                    device_id=y_nbr, device_id_type=pl.DeviceIdType.MESH,
                )
                d.start()
                fwd.append(d)
            recv_desc(ry, a, dyr.at[a]).wait_recv()
            if a >= nc2:
                sl = pl.ds(ry * qsize + a * ch, ch)
                d = pltpu.make_async_remote_copy(
                    src_ref=out_ref.at[sl], dst_ref=out_ref.at[sl],
                    send_sem=fzs.at[a - nc2], recv_sem=fzr.at[a - nc2],
                    device_id=z_nbr, device_id_type=pl.DeviceIdType.MESH,
                )
                d.start()
                fwd.append(d)

        xq_load.wait()

        st, dy, dz = [], [], []
        for c in range(NC):
            sl = pl.ds(c * ch, ch)
            p1[c].wait_recv()
            recv_ref[sl, :] = recv_ref[sl, :] + xq_ref[sl, :]
            s = pltpu.make_async_copy(
                recv_ref.at[sl], out_ref.at[pl.ds(base + c * ch, ch)],
                sts.at[c])
            s.start()
            st.append(s)
            for lst, sems_s, sems_r, nbr in (
                    (dy, dys, dyr, y_nbr), (dz, dzs, dzr, z_nbr)):
                d = pltpu.make_async_remote_copy(
                    src_ref=recv_ref.at[sl],
                    dst_ref=out_ref.at[pl.ds(base + c * ch, ch)],
                    send_sem=sems_s.at[c],
                    recv_sem=sems_r.at[c],
                    device_id=nbr,
                    device_id_type=pl.DeviceIdType.MESH,
                )
                d.start()
                lst.append(d)
            if c >= 2:
                process_arrival(c - 2)

        process_arrival(NC - 2)
        process_arrival(NC - 1)

        for c in range(nc2):
            recv_desc(diag, c, fyr.at[c]).wait_recv()
        for c in range(nc2, NC):
            recv_desc(diag, c, fzr.at[c - nc2]).wait_recv()

        for d in p1:
            d.wait_send()
        for d in st:
            d.wait()
        for d in dy + dz + fwd:
            d.wait_send()

    return pl.pallas_call(
        body,
        out_shape=jax.ShapeDtypeStruct((m, n), jnp.float32),
        in_specs=[pl.BlockSpec(memory_space=pltpu.MemorySpace.HBM)],
        out_specs=pl.BlockSpec(memory_space=pltpu.MemorySpace.HBM),
        scratch_shapes=[
            pltpu.VMEM((qsize, n), jnp.float32),
            pltpu.VMEM((qsize, n), jnp.float32),
            pltpu.SemaphoreType.DMA((NC,)),
            pltpu.SemaphoreType.DMA((NC,)),
            pltpu.SemaphoreType.DMA((NC,)),
            pltpu.SemaphoreType.DMA((NC,)),
            pltpu.SemaphoreType.DMA((NC,)),
            pltpu.SemaphoreType.DMA((NC,)),
            pltpu.SemaphoreType.DMA((NC // 2,)),
            pltpu.SemaphoreType.DMA((NC // 2,)),
            pltpu.SemaphoreType.DMA((NC // 2,)),
            pltpu.SemaphoreType.DMA((NC // 2,)),
            pltpu.SemaphoreType.DMA((NC,)),
            pltpu.SemaphoreType.DMA,
        ],
        compiler_params=pltpu.CompilerParams(collective_id=0),
    )(x)


# device time: 369369 ns/iter; 1.0356x vs baseline; 1.0356x over previous
import jax
import jax.numpy as jnp
from jax import lax
from jax.experimental import pallas as pl
from jax.experimental.pallas import tpu as pltpu

NC = 4


def kernel(x):
    m, n = x.shape
    qsize = m // 4
    ch = qsize // NC
    nc2 = NC // 2

    def body(x_ref, out_ref, recv_ref, xq_ref,
             p1s, p1r, dys, dyr, dzs, dzr, fys, fyr, fzs, fzr, sts, xls):
        my_x = lax.axis_index("x")
        my_y = lax.axis_index("y")
        my_z = lax.axis_index("z")
        r = 2 * my_y + my_z
        ry = r ^ 2
        rz = r ^ 1
        diag = r ^ 3
        base = r * qsize
        x_nbr = (1 - my_x, my_y, my_z)
        y_nbr = (my_x, 1 - my_y, my_z)
        z_nbr = (my_x, my_y, 1 - my_z)

        barrier_sem = pltpu.get_barrier_semaphore()
        for nbr in (x_nbr, y_nbr, z_nbr):
            pl.semaphore_signal(
                barrier_sem, inc=1, device_id=nbr,
                device_id_type=pl.DeviceIdType.MESH,
            )
        pl.semaphore_wait(barrier_sem, 3)

        xq_load = pltpu.make_async_copy(
            x_ref.at[pl.ds(base, qsize)], xq_ref, xls)
        xq_load.start()

        p1 = []
        for c in range(NC):
            d = pltpu.make_async_remote_copy(
                src_ref=x_ref.at[pl.ds(base + c * ch, ch)],
                dst_ref=recv_ref.at[pl.ds(c * ch, ch)],
                send_sem=p1s.at[c],
                recv_sem=p1r.at[c],
                device_id=x_nbr,
                device_id_type=pl.DeviceIdType.MESH,
            )
            d.start()
            p1.append(d)

        def recv_desc(qidx, c, sems_r):
            sl = pl.ds(qidx * qsize + c * ch, ch)
            return pltpu.make_async_remote_copy(
                src_ref=out_ref.at[sl], dst_ref=out_ref.at[sl],
                send_sem=p1s.at[0],
                recv_sem=sems_r,
                device_id=x_nbr,
                device_id_type=pl.DeviceIdType.MESH,
            )

        fwd = []

        def process_arrival(a):
            recv_desc(rz, a, dzr.at[a]).wait_recv()
            if a < nc2:
                sl = pl.ds(rz * qsize + a * ch, ch)
                d = pltpu.make_async_remote_copy(
                    src_ref=out_ref.at[sl], dst_ref=out_ref.at[sl],
                    send_sem=fys.at[a], recv_sem=fyr.at[a],
                    device_id=y_nbr, device_id_type=pl.DeviceIdType.MESH,
                )
                d.start()
                fwd.append(d)
            recv_desc(ry, a, dyr.at[a]).wait_recv()
            if a >= nc2:
                sl = pl.ds(ry * qsize + a * ch, ch)
                d = pltpu.make_async_remote_copy(
                    src_ref=out_ref.at[sl], dst_ref=out_ref.at[sl],
                    send_sem=fzs.at[a - nc2], recv_sem=fzr.at[a - nc2],
                    device_id=z_nbr, device_id_type=pl.DeviceIdType.MESH,
                )
                d.start()
                fwd.append(d)

        xq_load.wait()

        st, dy, dz = [], [], []
        for c in range(NC):
            sl = pl.ds(c * ch, ch)
            p1[c].wait_recv()
            recv_ref[sl, :] = recv_ref[sl, :] + xq_ref[sl, :]
            s = pltpu.make_async_copy(
                recv_ref.at[sl], out_ref.at[pl.ds(base + c * ch, ch)],
                sts.at[c])
            s.start()
            st.append(s)
            for lst, sems_s, sems_r, nbr in (
                    (dy, dys, dyr, y_nbr), (dz, dzs, dzr, z_nbr)):
                d = pltpu.make_async_remote_copy(
                    src_ref=recv_ref.at[sl],
                    dst_ref=out_ref.at[pl.ds(base + c * ch, ch)],
                    send_sem=sems_s.at[c],
                    recv_sem=sems_r.at[c],
                    device_id=nbr,
                    device_id_type=pl.DeviceIdType.MESH,
                )
                d.start()
                lst.append(d)
            if c >= 2:
                process_arrival(c - 2)

        process_arrival(NC - 2)
        process_arrival(NC - 1)

        for c in range(nc2):
            recv_desc(diag, c, fyr.at[c]).wait_recv()
        for c in range(nc2, NC):
            recv_desc(diag, c, fzr.at[c - nc2]).wait_recv()

        for d in p1:
            d.wait_send()
        for d in st:
            d.wait()
        for d in dy + dz + fwd:
            d.wait_send()

    return pl.pallas_call(
        body,
        out_shape=jax.ShapeDtypeStruct((m, n), jnp.float32),
        in_specs=[pl.BlockSpec(memory_space=pltpu.MemorySpace.HBM)],
        out_specs=pl.BlockSpec(memory_space=pltpu.MemorySpace.HBM),
        scratch_shapes=[
            pltpu.VMEM((qsize, n), jnp.float32),
            pltpu.VMEM((qsize, n), jnp.float32),
            pltpu.SemaphoreType.DMA((NC,)),
            pltpu.SemaphoreType.DMA((NC,)),
            pltpu.SemaphoreType.DMA((NC,)),
            pltpu.SemaphoreType.DMA((NC,)),
            pltpu.SemaphoreType.DMA((NC,)),
            pltpu.SemaphoreType.DMA((NC,)),
            pltpu.SemaphoreType.DMA((NC // 2,)),
            pltpu.SemaphoreType.DMA((NC // 2,)),
            pltpu.SemaphoreType.DMA((NC // 2,)),
            pltpu.SemaphoreType.DMA((NC // 2,)),
            pltpu.SemaphoreType.DMA((NC,)),
            pltpu.SemaphoreType.DMA,
        ],
        compiler_params=pltpu.CompilerParams(collective_id=0),
    )(x)


# device time: 369360 ns/iter; 1.0356x vs baseline; 1.0000x over previous
import jax
import jax.numpy as jnp
from jax import lax
from jax.experimental import pallas as pl
from jax.experimental.pallas import tpu as pltpu

NC = 4


def kernel(x):
    m, n = x.shape
    qsize = m // 4
    ch = qsize // NC
    nc2 = NC // 2

    def body(x_ref, out_ref, recv_ref, xq_ref,
             p1s, p1r, dys, dyr, dzs, dzr, fys, fyr, fzs, fzr, sts, xls):
        my_x = lax.axis_index("x")
        my_y = lax.axis_index("y")
        my_z = lax.axis_index("z")
        r = 2 * my_y + my_z
        ry = r ^ 2
        rz = r ^ 1
        diag = r ^ 3
        base = r * qsize
        x_nbr = (1 - my_x, my_y, my_z)
        y_nbr = (my_x, 1 - my_y, my_z)
        z_nbr = (my_x, my_y, 1 - my_z)

        barrier_sem = pltpu.get_barrier_semaphore()
        for nbr in (x_nbr, y_nbr, z_nbr):
            pl.semaphore_signal(
                barrier_sem, inc=1, device_id=nbr,
                device_id_type=pl.DeviceIdType.MESH,
            )
        pl.semaphore_wait(barrier_sem, 3)

        xq_load = pltpu.make_async_copy(
            x_ref.at[pl.ds(base, qsize)], xq_ref, xls)
        xq_load.start()

        p1 = []
        for c in range(NC):
            d = pltpu.make_async_remote_copy(
                src_ref=x_ref.at[pl.ds(base + c * ch, ch)],
                dst_ref=recv_ref.at[pl.ds(c * ch, ch)],
                send_sem=p1s.at[c],
                recv_sem=p1r.at[c],
                device_id=x_nbr,
                device_id_type=pl.DeviceIdType.MESH,
            )
            d.start()
            p1.append(d)

        def recv_desc(qidx, c, sems_r):
            sl = pl.ds(qidx * qsize + c * ch, ch)
            return pltpu.make_async_remote_copy(
                src_ref=out_ref.at[sl], dst_ref=out_ref.at[sl],
                send_sem=p1s.at[0],
                recv_sem=sems_r,
                device_id=x_nbr,
                device_id_type=pl.DeviceIdType.MESH,
            )

        fwd = []

        def process_arrival(a):
            recv_desc(rz, a, dzr.at[a]).wait_recv()
            if a < nc2:
                sl = pl.ds(rz * qsize + a * ch, ch)
                d = pltpu.make_async_remote_copy(
                    src_ref=out_ref.at[sl], dst_ref=out_ref.at[sl],
                    send_sem=fys.at[a], recv_sem=fyr.at[a],
                    device_id=y_nbr, device_id_type=pl.DeviceIdType.MESH,
                )
                d.start()
                fwd.append(d)
            recv_desc(ry, a, dyr.at[a]).wait_recv()
            if a >= nc2:
                sl = pl.ds(ry * qsize + a * ch, ch)
                d = pltpu.make_async_remote_copy(
                    src_ref=out_ref.at[sl], dst_ref=out_ref.at[sl],
                    send_sem=fzs.at[a - nc2], recv_sem=fzr.at[a - nc2],
                    device_id=z_nbr, device_id_type=pl.DeviceIdType.MESH,
                )
                d.start()
                fwd.append(d)

        xq_load.wait()

        dy, dz = [], []
        for c in range(NC):
            sl = pl.ds(c * ch, ch)
            p1[c].wait_recv()
            recv_ref[sl, :] = recv_ref[sl, :] + xq_ref[sl, :]
            for lst, sems_s, sems_r, nbr in (
                    (dy, dys, dyr, y_nbr), (dz, dzs, dzr, z_nbr)):
                d = pltpu.make_async_remote_copy(
                    src_ref=recv_ref.at[sl],
                    dst_ref=out_ref.at[pl.ds(base + c * ch, ch)],
                    send_sem=sems_s.at[c],
                    recv_sem=sems_r.at[c],
                    device_id=nbr,
                    device_id_type=pl.DeviceIdType.MESH,
                )
                d.start()
                lst.append(d)
            if c >= 2:
                process_arrival(c - 2)

        st = pltpu.make_async_copy(
            recv_ref, out_ref.at[pl.ds(base, qsize)], sts.at[0])
        st.start()

        process_arrival(NC - 2)
        process_arrival(NC - 1)

        for c in range(nc2):
            recv_desc(diag, c, fyr.at[c]).wait_recv()
        for c in range(nc2, NC):
            recv_desc(diag, c, fzr.at[c - nc2]).wait_recv()

        for d in p1:
            d.wait_send()
        st.wait()
        for d in dy + dz + fwd:
            d.wait_send()

    return pl.pallas_call(
        body,
        out_shape=jax.ShapeDtypeStruct((m, n), jnp.float32),
        in_specs=[pl.BlockSpec(memory_space=pltpu.MemorySpace.HBM)],
        out_specs=pl.BlockSpec(memory_space=pltpu.MemorySpace.HBM),
        scratch_shapes=[
            pltpu.VMEM((qsize, n), jnp.float32),
            pltpu.VMEM((qsize, n), jnp.float32),
            pltpu.SemaphoreType.DMA((NC,)),
            pltpu.SemaphoreType.DMA((NC,)),
            pltpu.SemaphoreType.DMA((NC,)),
            pltpu.SemaphoreType.DMA((NC,)),
            pltpu.SemaphoreType.DMA((NC,)),
            pltpu.SemaphoreType.DMA((NC,)),
            pltpu.SemaphoreType.DMA((NC // 2,)),
            pltpu.SemaphoreType.DMA((NC // 2,)),
            pltpu.SemaphoreType.DMA((NC // 2,)),
            pltpu.SemaphoreType.DMA((NC // 2,)),
            pltpu.SemaphoreType.DMA((1,)),
            pltpu.SemaphoreType.DMA,
        ],
        compiler_params=pltpu.CompilerParams(collective_id=0),
    )(x)
